# baseline (speedup 1.0000x reference)
import jax
import jax.numpy as jnp
from jax.experimental import pallas as pl
from jax.experimental.pallas import tpu as pltpu

N = 1024
OBS = 128
HID = 256
H = 256
K = 8
R = N // K


def _encoder_body(obs_ref, hid_ref, mat_ref,
                  obs_cW_ref, obs_cb_ref, obs_f1W_ref, obs_f1b_ref,
                  obs_f2W_ref, obs_f2b_ref,
                  hid_cW_ref, hid_cb_ref, hid_f1W_ref, hid_f1b_ref,
                  hid_f2W_ref, hid_f2b_ref,
                  gru_Wih_ref, gru_Whh_ref, gru_bih_ref, gru_bhh_ref,
                  enc_W_ref, enc_b_ref,
                  latent_ref, next_hid_ref,
                  deg_ref, dinv_ref, agg_o_ref, agg_h_ref):
    p = pl.program_id(0)
    k = pl.program_id(1)
    mf = mat_ref[...].astype(jnp.float32)

    @pl.when(p == 0)
    def _degree_phase():
        ones = jnp.ones((R, 1), jnp.float32)
        part = jax.lax.dot_general(
            mf, ones, (((0,), (0,)), ((), ())),
            preferred_element_type=jnp.float32)

        @pl.when(k == 0)
        def _():
            deg_ref[...] = part + 1.0

        @pl.when(k != 0)
        def _():
            deg_ref[...] += part

    @pl.when(p == 1)
    def _matmul_phase():
        @pl.when(k == 0)
        def _():
            dinv_ref[...] = jax.lax.rsqrt(deg_ref[...])
            agg_o_ref[...] = jnp.zeros((N, H), jnp.float32)
            agg_h_ref[...] = jnp.zeros((N, H), jnp.float32)

        dchunk = dinv_ref[pl.ds(k * R, R), :]
        s_o = dchunk * jnp.dot(obs_ref[...], obs_cW_ref[...],
                               preferred_element_type=jnp.float32)
        s_h = dchunk * jnp.dot(hid_ref[...], hid_cW_ref[...],
                               preferred_element_type=jnp.float32)
        agg_o_ref[...] += jax.lax.dot_general(
            mf, s_o, (((0,), (0,)), ((), ())),
            preferred_element_type=jnp.float32)
        agg_h_ref[...] += jax.lax.dot_general(
            mf, s_h, (((0,), (0,)), ((), ())),
            preferred_element_type=jnp.float32)
        agg_o_ref[pl.ds(k * R, R), :] += s_o
        agg_h_ref[pl.ds(k * R, R), :] += s_h

        @pl.when(k == K - 1)
        def _tail():
            dinv = dinv_ref[...]

            def mlp(agg, cb, f1W, f1b, f2W, f2b):
                h = jnp.maximum(dinv * agg + cb, 0.0)
                h = jnp.maximum(jnp.dot(h, f1W,
                                        preferred_element_type=jnp.float32)
                                + f1b, 0.0)
                return jnp.dot(h, f2W,
                               preferred_element_type=jnp.float32) + f2b

            phi = mlp(agg_o_ref[...], obs_cb_ref[...],
                      obs_f1W_ref[...], obs_f1b_ref[...],
                      obs_f2W_ref[...], obs_f2b_ref[...])
            psi = mlp(agg_h_ref[...], hid_cb_ref[...],
                      hid_f1W_ref[...], hid_f1b_ref[...],
                      hid_f2W_ref[...], hid_f2b_ref[...])

            gi = jax.lax.dot_general(
                phi, gru_Wih_ref[...], (((1,), (1,)), ((), ())),
                preferred_element_type=jnp.float32) + gru_bih_ref[...]
            gh = jax.lax.dot_general(
                psi, gru_Whh_ref[...], (((1,), (1,)), ((), ())),
                preferred_element_type=jnp.float32) + gru_bhh_ref[...]
            r = jax.nn.sigmoid(gi[:, :HID] + gh[:, :HID])
            z = jax.nn.sigmoid(gi[:, HID:2 * HID] + gh[:, HID:2 * HID])
            n = jnp.tanh(gi[:, 2 * HID:] + r * gh[:, 2 * HID:])
            next_hid = (1.0 - z) * n + z * psi

            latent_ref[...] = jnp.dot(
                next_hid, enc_W_ref[...],
                preferred_element_type=jnp.float32) + enc_b_ref[...]
            next_hid_ref[...] = next_hid


def kernel(obs, hidden_states, matrix,
           obs_cW, obs_cb, obs_f1W, obs_f1b, obs_f2W, obs_f2b,
           hid_cW, hid_cb, hid_f1W, hid_f1b, hid_f2W, hid_f2b,
           gru_Wih, gru_Whh, gru_bih, gru_bhh,
           enc_W, enc_b):
    chunk = lambda p, k: (k, 0)
    whole = lambda p, k: (0, 0)
    vec = lambda p, k: (0,)
    latent, next_hid = pl.pallas_call(
        _encoder_body,
        grid=(2, K),
        in_specs=[
            pl.BlockSpec((R, OBS), chunk),
            pl.BlockSpec((R, HID), chunk),
            pl.BlockSpec((R, N), chunk),
            pl.BlockSpec((OBS, H), whole),
            pl.BlockSpec((H,), vec),
            pl.BlockSpec((H, H), whole),
            pl.BlockSpec((H,), vec),
            pl.BlockSpec((H, OBS), whole),
            pl.BlockSpec((OBS,), vec),
            pl.BlockSpec((HID, H), whole),
            pl.BlockSpec((H,), vec),
            pl.BlockSpec((H, H), whole),
            pl.BlockSpec((H,), vec),
            pl.BlockSpec((H, HID), whole),
            pl.BlockSpec((HID,), vec),
            pl.BlockSpec((3 * HID, OBS), whole),
            pl.BlockSpec((3 * HID, HID), whole),
            pl.BlockSpec((3 * HID,), vec),
            pl.BlockSpec((3 * HID,), vec),
            pl.BlockSpec((HID, H), whole),
            pl.BlockSpec((H,), vec),
        ],
        out_specs=(
            pl.BlockSpec((N, H), whole),
            pl.BlockSpec((N, HID), whole),
        ),
        out_shape=(
            jax.ShapeDtypeStruct((N, H), jnp.float32),
            jax.ShapeDtypeStruct((N, HID), jnp.float32),
        ),
        scratch_shapes=[
            pltpu.VMEM((N, 1), jnp.float32),
            pltpu.VMEM((N, 1), jnp.float32),
            pltpu.VMEM((N, H), jnp.float32),
            pltpu.VMEM((N, H), jnp.float32),
        ],
    )(obs, hidden_states, matrix,
      obs_cW, obs_cb, obs_f1W, obs_f1b, obs_f2W, obs_f2b,
      hid_cW, hid_cb, hid_f1W, hid_f1b, hid_f2W, hid_f2b,
      gru_Wih, gru_Whh, gru_bih, gru_bhh,
      enc_W, enc_b)
    return (latent, next_hid)


# speedup vs baseline: 1.2837x; 1.2837x over previous
import jax
import jax.numpy as jnp
from jax.experimental import pallas as pl
from jax.experimental.pallas import tpu as pltpu

N = 1024
OBS = 128
HID = 256
H = 256
K = 4
C = N // K


def _encoder_body(obs_ref, hid_ref, mat_ref,
                  obs_cW_ref, obs_cb_ref, obs_f1W_ref, obs_f1b_ref,
                  obs_f2W_ref, obs_f2b_ref,
                  hid_cW_ref, hid_cb_ref, hid_f1W_ref, hid_f1b_ref,
                  hid_f2W_ref, hid_f2b_ref,
                  gru_Wih_ref, gru_Whh_ref, gru_bih_ref, gru_bhh_ref,
                  enc_W_ref, enc_b_ref,
                  latent_ref, next_hid_ref,
                  dinv_ref, s_o_ref, s_h_ref):
    p = pl.program_id(0)
    k = pl.program_id(1)
    mf = mat_ref[...].astype(jnp.float32)

    @pl.when(p == 0)
    def _degree_phase():
        ones = jnp.ones((N, 1), jnp.float32)
        deg = jax.lax.dot_general(
            mf, ones, (((0,), (0,)), ((), ())),
            preferred_element_type=jnp.float32) + 1.0
        dinv_ref[pl.ds(k * C, C), :] = jax.lax.rsqrt(deg)

    @pl.when(p == 1)
    def _compute_phase():
        @pl.when(k == 0)
        def _():
            dinv = dinv_ref[...]
            s_o_ref[...] = dinv * jnp.dot(obs_ref[...], obs_cW_ref[...],
                                          preferred_element_type=jnp.float32)
            s_h_ref[...] = dinv * jnp.dot(hid_ref[...], hid_cW_ref[...],
                                          preferred_element_type=jnp.float32)

        dchunk = dinv_ref[pl.ds(k * C, C), :]

        def gcn_mlp(s_ref, cb, f1W, f1b, f2W, f2b):
            slab = jax.lax.dot_general(
                mf, s_ref[...], (((0,), (0,)), ((), ())),
                preferred_element_type=jnp.float32)
            slab += s_ref[pl.ds(k * C, C), :]
            h = jnp.maximum(dchunk * slab + cb, 0.0)
            h = jnp.maximum(jnp.dot(h, f1W,
                                    preferred_element_type=jnp.float32)
                            + f1b, 0.0)
            return jnp.dot(h, f2W, preferred_element_type=jnp.float32) + f2b

        phi = gcn_mlp(s_o_ref, obs_cb_ref[...],
                      obs_f1W_ref[...], obs_f1b_ref[...],
                      obs_f2W_ref[...], obs_f2b_ref[...])
        psi = gcn_mlp(s_h_ref, hid_cb_ref[...],
                      hid_f1W_ref[...], hid_f1b_ref[...],
                      hid_f2W_ref[...], hid_f2b_ref[...])

        gi = jax.lax.dot_general(
            phi, gru_Wih_ref[...], (((1,), (1,)), ((), ())),
            preferred_element_type=jnp.float32) + gru_bih_ref[...]
        gh = jax.lax.dot_general(
            psi, gru_Whh_ref[...], (((1,), (1,)), ((), ())),
            preferred_element_type=jnp.float32) + gru_bhh_ref[...]
        r = jax.nn.sigmoid(gi[:, :HID] + gh[:, :HID])
        z = jax.nn.sigmoid(gi[:, HID:2 * HID] + gh[:, HID:2 * HID])
        n = jnp.tanh(gi[:, 2 * HID:] + r * gh[:, 2 * HID:])
        next_hid = (1.0 - z) * n + z * psi

        latent_ref[...] = jnp.dot(next_hid, enc_W_ref[...],
                                  preferred_element_type=jnp.float32) + enc_b_ref[...]
        next_hid_ref[...] = next_hid


def kernel(obs, hidden_states, matrix,
           obs_cW, obs_cb, obs_f1W, obs_f1b, obs_f2W, obs_f2b,
           hid_cW, hid_cb, hid_f1W, hid_f1b, hid_f2W, hid_f2b,
           gru_Wih, gru_Whh, gru_bih, gru_bhh,
           enc_W, enc_b):
    colchunk = lambda p, k: (0, k)
    whole = lambda p, k: (0, 0)
    vec = lambda p, k: (0,)
    outmap = lambda p, k: (jax.lax.select(p == 1, k, 0), 0)
    latent, next_hid = pl.pallas_call(
        _encoder_body,
        grid=(2, K),
        in_specs=[
            pl.BlockSpec((N, OBS), whole),
            pl.BlockSpec((N, HID), whole),
            pl.BlockSpec((N, C), colchunk),
            pl.BlockSpec((OBS, H), whole),
            pl.BlockSpec((H,), vec),
            pl.BlockSpec((H, H), whole),
            pl.BlockSpec((H,), vec),
            pl.BlockSpec((H, OBS), whole),
            pl.BlockSpec((OBS,), vec),
            pl.BlockSpec((HID, H), whole),
            pl.BlockSpec((H,), vec),
            pl.BlockSpec((H, H), whole),
            pl.BlockSpec((H,), vec),
            pl.BlockSpec((H, HID), whole),
            pl.BlockSpec((HID,), vec),
            pl.BlockSpec((3 * HID, OBS), whole),
            pl.BlockSpec((3 * HID, HID), whole),
            pl.BlockSpec((3 * HID,), vec),
            pl.BlockSpec((3 * HID,), vec),
            pl.BlockSpec((HID, H), whole),
            pl.BlockSpec((H,), vec),
        ],
        out_specs=(
            pl.BlockSpec((C, H), outmap),
            pl.BlockSpec((C, HID), outmap),
        ),
        out_shape=(
            jax.ShapeDtypeStruct((N, H), jnp.float32),
            jax.ShapeDtypeStruct((N, HID), jnp.float32),
        ),
        scratch_shapes=[
            pltpu.VMEM((N, 1), jnp.float32),
            pltpu.VMEM((N, H), jnp.float32),
            pltpu.VMEM((N, H), jnp.float32),
        ],
    )(obs, hidden_states, matrix,
      obs_cW, obs_cb, obs_f1W, obs_f1b, obs_f2W, obs_f2b,
      hid_cW, hid_cb, hid_f1W, hid_f1b, hid_f2W, hid_f2b,
      gru_Wih, gru_Whh, gru_bih, gru_bhh,
      enc_W, enc_b)
    return (latent, next_hid)


# speedup vs baseline: 2.0164x; 1.5707x over previous
import jax
import jax.numpy as jnp
from jax.experimental import pallas as pl
from jax.experimental.pallas import tpu as pltpu

N = 1024
OBS = 128
HID = 256
H = 256
C = N // 2


def _encoder_body(obs_ref, hid_ref, mat_ref,
                  obs_cW_ref, obs_cb_ref, obs_f1W_ref, obs_f1b_ref,
                  obs_f2W_ref, obs_f2b_ref,
                  hid_cW_ref, hid_cb_ref, hid_f1W_ref, hid_f1b_ref,
                  hid_f2W_ref, hid_f2b_ref,
                  gru_Wih_ref, gru_Whh_ref, gru_bih_ref, gru_bhh_ref,
                  enc_W_ref, enc_b_ref,
                  latent_ref, next_hid_ref,
                  dinv_ref, mfA_ref):
    k = pl.program_id(0)
    mf = mat_ref[...].astype(jnp.float32)

    ones = jnp.ones((N, 1), jnp.float32)
    deg = jax.lax.dot_general(
        mf, ones, (((0,), (0,)), ((), ())),
        preferred_element_type=jnp.float32) + 1.0
    dinv_ref[pl.ds(k * C, C), :] = jax.lax.rsqrt(deg)

    @pl.when(k == 0)
    def _park():
        mfA_ref[...] = mf

    @pl.when(k == 1)
    def _tail():
        dinv = dinv_ref[...]

        def gcn_mlp(x, cW, cb, f1W, f1b, f2W, f2b):
            s = dinv * jnp.dot(x, cW, preferred_element_type=jnp.float32)
            aggA = jax.lax.dot_general(
                mfA_ref[...], s, (((0,), (0,)), ((), ())),
                preferred_element_type=jnp.float32)
            aggB = jax.lax.dot_general(
                mf, s, (((0,), (0,)), ((), ())),
                preferred_element_type=jnp.float32)
            agg = jnp.concatenate([aggA, aggB], axis=0) + s
            h = jnp.maximum(dinv * agg + cb, 0.0)
            h = jnp.maximum(jnp.dot(h, f1W,
                                    preferred_element_type=jnp.float32)
                            + f1b, 0.0)
            return jnp.dot(h, f2W, preferred_element_type=jnp.float32) + f2b

        phi = gcn_mlp(obs_ref[...], obs_cW_ref[...], obs_cb_ref[...],
                      obs_f1W_ref[...], obs_f1b_ref[...],
                      obs_f2W_ref[...], obs_f2b_ref[...])
        psi = gcn_mlp(hid_ref[...], hid_cW_ref[...], hid_cb_ref[...],
                      hid_f1W_ref[...], hid_f1b_ref[...],
                      hid_f2W_ref[...], hid_f2b_ref[...])

        gi = jax.lax.dot_general(
            phi, gru_Wih_ref[...], (((1,), (1,)), ((), ())),
            preferred_element_type=jnp.float32) + gru_bih_ref[...]
        gh = jax.lax.dot_general(
            psi, gru_Whh_ref[...], (((1,), (1,)), ((), ())),
            preferred_element_type=jnp.float32) + gru_bhh_ref[...]
        r = jax.nn.sigmoid(gi[:, :HID] + gh[:, :HID])
        z = jax.nn.sigmoid(gi[:, HID:2 * HID] + gh[:, HID:2 * HID])
        n = jnp.tanh(gi[:, 2 * HID:] + r * gh[:, 2 * HID:])
        next_hid = (1.0 - z) * n + z * psi

        latent_ref[...] = jnp.dot(next_hid, enc_W_ref[...],
                                  preferred_element_type=jnp.float32) + enc_b_ref[...]
        next_hid_ref[...] = next_hid


def kernel(obs, hidden_states, matrix,
           obs_cW, obs_cb, obs_f1W, obs_f1b, obs_f2W, obs_f2b,
           hid_cW, hid_cb, hid_f1W, hid_f1b, hid_f2W, hid_f2b,
           gru_Wih, gru_Whh, gru_bih, gru_bhh,
           enc_W, enc_b):
    colchunk = lambda k: (0, k)
    whole = lambda k: (0, 0)
    vec = lambda k: (0,)
    latent, next_hid = pl.pallas_call(
        _encoder_body,
        grid=(2,),
        in_specs=[
            pl.BlockSpec((N, OBS), whole),
            pl.BlockSpec((N, HID), whole),
            pl.BlockSpec((N, C), colchunk),
            pl.BlockSpec((OBS, H), whole),
            pl.BlockSpec((H,), vec),
            pl.BlockSpec((H, H), whole),
            pl.BlockSpec((H,), vec),
            pl.BlockSpec((H, OBS), whole),
            pl.BlockSpec((OBS,), vec),
            pl.BlockSpec((HID, H), whole),
            pl.BlockSpec((H,), vec),
            pl.BlockSpec((H, H), whole),
            pl.BlockSpec((H,), vec),
            pl.BlockSpec((H, HID), whole),
            pl.BlockSpec((HID,), vec),
            pl.BlockSpec((3 * HID, OBS), whole),
            pl.BlockSpec((3 * HID, HID), whole),
            pl.BlockSpec((3 * HID,), vec),
            pl.BlockSpec((3 * HID,), vec),
            pl.BlockSpec((HID, H), whole),
            pl.BlockSpec((H,), vec),
        ],
        out_specs=(
            pl.BlockSpec((N, H), whole),
            pl.BlockSpec((N, HID), whole),
        ),
        out_shape=(
            jax.ShapeDtypeStruct((N, H), jnp.float32),
            jax.ShapeDtypeStruct((N, HID), jnp.float32),
        ),
        scratch_shapes=[
            pltpu.VMEM((N, 1), jnp.float32),
            pltpu.VMEM((N, C), jnp.float32),
        ],
    )(obs, hidden_states, matrix,
      obs_cW, obs_cb, obs_f1W, obs_f1b, obs_f2W, obs_f2b,
      hid_cW, hid_cb, hid_f1W, hid_f1b, hid_f2W, hid_f2b,
      gru_Wih, gru_Whh, gru_bih, gru_bhh,
      enc_W, enc_b)
    return (latent, next_hid)


# speedup vs baseline: 2.0373x; 1.0104x over previous
import jax
import jax.numpy as jnp
from jax.experimental import pallas as pl
from jax.experimental.pallas import tpu as pltpu

N = 1024
OBS = 128
HID = 256
H = 256
C = N // 2


def _encoder_body(obs_ref, hid_ref, mat_ref,
                  obs_cW_ref, obs_cb_ref, obs_f1W_ref, obs_f1b_ref,
                  obs_f2W_ref, obs_f2b_ref,
                  hid_cW_ref, hid_cb_ref, hid_f1W_ref, hid_f1b_ref,
                  hid_f2W_ref, hid_f2b_ref,
                  gru_Wih_ref, gru_Whh_ref, gru_bih_ref, gru_bhh_ref,
                  enc_W_ref, enc_b_ref,
                  latent_ref, next_hid_ref,
                  dinv_ref, mfA_ref):
    k = pl.program_id(0)
    mf = mat_ref[...].astype(jnp.bfloat16)

    def dot16(a, b):
        return jnp.dot(a.astype(jnp.bfloat16), b.astype(jnp.bfloat16),
                       preferred_element_type=jnp.float32)

    def dot16_t(a, b):
        return jax.lax.dot_general(
            a.astype(jnp.bfloat16), b.astype(jnp.bfloat16),
            (((0,), (0,)), ((), ())), preferred_element_type=jnp.float32)

    ones = jnp.ones((N, 1), jnp.bfloat16)
    deg = dot16_t(mf, ones) + 1.0
    dinv_ref[pl.ds(k * C, C), :] = jax.lax.rsqrt(deg)

    @pl.when(k == 0)
    def _park():
        mfA_ref[...] = mf

    @pl.when(k == 1)
    def _tail():
        dinv = dinv_ref[...]

        def gcn_mlp(x, cW, cb, f1W, f1b, f2W, f2b):
            s = dinv * dot16(x, cW)
            agg = jnp.concatenate(
                [dot16_t(mfA_ref[...], s), dot16_t(mf, s)],
                axis=0) + s
            h = jnp.maximum(dinv * agg + cb, 0.0)
            h = jnp.maximum(dot16(h, f1W) + f1b, 0.0)
            return dot16(h, f2W) + f2b

        phi = gcn_mlp(obs_ref[...], obs_cW_ref[...], obs_cb_ref[...],
                      obs_f1W_ref[...], obs_f1b_ref[...],
                      obs_f2W_ref[...], obs_f2b_ref[...])
        psi = gcn_mlp(hid_ref[...], hid_cW_ref[...], hid_cb_ref[...],
                      hid_f1W_ref[...], hid_f1b_ref[...],
                      hid_f2W_ref[...], hid_f2b_ref[...])

        def dot16_rt(a, b):
            return jax.lax.dot_general(
                a.astype(jnp.bfloat16), b.astype(jnp.bfloat16),
                (((1,), (1,)), ((), ())), preferred_element_type=jnp.float32)

        gi = dot16_rt(phi, gru_Wih_ref[...]) + gru_bih_ref[...]
        gh = dot16_rt(psi, gru_Whh_ref[...]) + gru_bhh_ref[...]
        r = jax.nn.sigmoid(gi[:, :HID] + gh[:, :HID])
        z = jax.nn.sigmoid(gi[:, HID:2 * HID] + gh[:, HID:2 * HID])
        n = jnp.tanh(gi[:, 2 * HID:] + r * gh[:, 2 * HID:])
        next_hid = (1.0 - z) * n + z * psi

        latent_ref[...] = dot16(next_hid, enc_W_ref[...]) + enc_b_ref[...]
        next_hid_ref[...] = next_hid


def kernel(obs, hidden_states, matrix,
           obs_cW, obs_cb, obs_f1W, obs_f1b, obs_f2W, obs_f2b,
           hid_cW, hid_cb, hid_f1W, hid_f1b, hid_f2W, hid_f2b,
           gru_Wih, gru_Whh, gru_bih, gru_bhh,
           enc_W, enc_b):
    colchunk = lambda k: (0, k)
    whole = lambda k: (0, 0)
    vec = lambda k: (0,)
    latent, next_hid = pl.pallas_call(
        _encoder_body,
        grid=(2,),
        in_specs=[
            pl.BlockSpec((N, OBS), whole),
            pl.BlockSpec((N, HID), whole),
            pl.BlockSpec((N, C), colchunk),
            pl.BlockSpec((OBS, H), whole),
            pl.BlockSpec((H,), vec),
            pl.BlockSpec((H, H), whole),
            pl.BlockSpec((H,), vec),
            pl.BlockSpec((H, OBS), whole),
            pl.BlockSpec((OBS,), vec),
            pl.BlockSpec((HID, H), whole),
            pl.BlockSpec((H,), vec),
            pl.BlockSpec((H, H), whole),
            pl.BlockSpec((H,), vec),
            pl.BlockSpec((H, HID), whole),
            pl.BlockSpec((HID,), vec),
            pl.BlockSpec((3 * HID, OBS), whole),
            pl.BlockSpec((3 * HID, HID), whole),
            pl.BlockSpec((3 * HID,), vec),
            pl.BlockSpec((3 * HID,), vec),
            pl.BlockSpec((HID, H), whole),
            pl.BlockSpec((H,), vec),
        ],
        out_specs=(
            pl.BlockSpec((N, H), whole),
            pl.BlockSpec((N, HID), whole),
        ),
        out_shape=(
            jax.ShapeDtypeStruct((N, H), jnp.float32),
            jax.ShapeDtypeStruct((N, HID), jnp.float32),
        ),
        scratch_shapes=[
            pltpu.VMEM((N, 1), jnp.float32),
            pltpu.VMEM((N, C), jnp.bfloat16),
        ],
    )(obs, hidden_states, matrix,
      obs_cW, obs_cb, obs_f1W, obs_f1b, obs_f2W, obs_f2b,
      hid_cW, hid_cb, hid_f1W, hid_f1b, hid_f2W, hid_f2b,
      gru_Wih, gru_Whh, gru_bih, gru_bhh,
      enc_W, enc_b)
    return (latent, next_hid)
